# baseline (speedup 1.0000x reference)
import functools

import jax
import jax.numpy as jnp
from jax import lax
from jax.experimental import pallas as pl
from jax.experimental.pallas import tpu as pltpu
from jax.experimental.pallas import tpu_sc as plsc

D_MODEL = 64
N_KEYS = 1000000
CHUNK = 20000
N_CHUNKS = N_KEYS // CHUNK
BLK = 2500
N_BLKS = N_KEYS // BLK
BLK_PER_CHUNK = CHUNK // BLK
K = 16
NSEL = 24

_NEG = float("-inf")
_IMAX = 2**31 - 1


def _sims_block(qnb, x):
    sumsq = jnp.sum(x * x, axis=1, keepdims=True)
    kn = x / (jnp.sqrt(sumsq) + 1e-8)
    knb = kn.astype(jnp.bfloat16)
    return lax.dot_general(
        qnb, knb, dimension_numbers=(((1,), (1,)), ((), ())),
        preferred_element_type=jnp.float32)


def _stage_a_body(qnb_ref, keys_ref, bmax_ref):
    qnb = qnb_ref[...]
    iota_b = lax.broadcasted_iota(jnp.int32, (1, BLK_PER_CHUNK), 1)
    bmax = jnp.zeros((1, BLK_PER_CHUNK), jnp.float32)
    for b in range(BLK_PER_CHUNK):
        x_b = keys_ref[pl.ds(b * BLK, BLK), :]
        mb = jnp.max(_sims_block(qnb, x_b))
        bmax = jnp.where(iota_b == b, mb, bmax)
    bmax_ref[...] = bmax.reshape(1, 1, BLK_PER_CHUNK)


def _stage_s_body(cmax_ref, sel_ref):
    work = cmax_ref[...]
    iota = lax.broadcasted_iota(jnp.int32, (1, N_BLKS), 1)
    iota_s = lax.broadcasted_iota(jnp.int32, (1, NSEL), 1)
    sel = jnp.zeros((1, NSEL), jnp.int32)
    for t in range(NSEL):
        m = jnp.max(work)
        pos = jnp.min(jnp.where(work == m, iota, _IMAX))
        sel = jnp.where(iota_s == t, pos, sel)
        work = jnp.where(iota == pos, _NEG, work)
    sel_ref[...] = sel


def _stage_b_body(sel_ref, qnb_ref, keys_ref, tv_ref, ti_ref, cand_scr):
    i = pl.program_id(0)
    x = keys_ref[...].reshape(BLK, D_MODEL)
    cand_scr[pl.ds(i, 1), :] = _sims_block(qnb_ref[...], x)

    @pl.when(i == NSEL - 1)
    def _():
        work = cand_scr[...]
        iota_sc = lax.broadcasted_iota(jnp.int32, (NSEL, 1), 0)
        sel_col = jnp.zeros((NSEL, 1), jnp.int32)
        for t in range(NSEL):
            sel_col = jnp.where(iota_sc == t, sel_ref[t], sel_col)
        gidx = (sel_col * BLK
                + lax.broadcasted_iota(jnp.int32, (NSEL, BLK), 1))
        iota_k = lax.broadcasted_iota(jnp.int32, (1, K), 1)
        vals_vec = jnp.zeros((1, K), jnp.float32)
        idx_vec = jnp.zeros((1, K), jnp.int32)
        for t in range(K):
            m = jnp.max(work)
            tie = work == m
            bidx = jnp.min(jnp.where(tie, gidx, _IMAX))
            vals_vec = jnp.where(iota_k == t, m, vals_vec)
            idx_vec = jnp.where(iota_k == t, bidx, idx_vec)
            work = jnp.where(tie & (gidx == bidx), _NEG, work)
        tv_ref[...] = vals_vec
        ti_ref[...] = idx_vec


def _stage_c_body(keys_hbm, vals_hbm, idx_hbm, q_hbm, morph_hbm, w_hbm,
                  b_hbm, out_hbm, idx_v, vals_v, rows_v, mq_v, w_v, b_v,
                  out_v, cvec_v, sem):
    core = lax.axis_index("c")
    sub = lax.axis_index("s")

    @pl.when((core == 0) & (sub == 0))
    def _():
        pltpu.sync_copy(idx_hbm, idx_v)
        pltpu.sync_copy(vals_hbm, vals_v.at[pl.ds(16, K)])
        pltpu.async_copy(keys_hbm.at[idx_v], rows_v, sem).wait()
        pltpu.sync_copy(w_hbm, w_v)
        pltpu.sync_copy(b_hbm, b_v)
        pltpu.sync_copy(morph_hbm, mq_v.at[pl.ds(16, D_MODEL)])
        pltpu.sync_copy(q_hbm, mq_v.at[pl.ds(16 + D_MODEL, D_MODEL)])

        def splat(ref, i):
            return plsc.load_gather(ref, [jnp.full((16,), 16 + i,
                                                   jnp.int32)])

        m = splat(vals_v, 0)
        for k in range(1, K):
            m = jnp.maximum(m, splat(vals_v, k))
        s = jnp.zeros((16,), jnp.float32)
        for k in range(K):
            s = s + jnp.exp(splat(vals_v, k) - m)

        nj = D_MODEL // 16
        cacc = [jnp.zeros((16,), jnp.float32) for _ in range(nj)]
        for k in range(K):
            wk = jnp.exp(splat(vals_v, k) - m) / s
            for j in range(nj):
                cacc[j] = cacc[j] + wk * rows_v[k, pl.ds(j * 16, 16)]
        for j in range(nj):
            out_v[pl.ds(j * 16, 16)] = cacc[j]
        pltpu.sync_copy(out_v, out_hbm)
        pltpu.sync_copy(out_hbm, cvec_v.at[pl.ds(16, D_MODEL)])

        oacc = [b_v[pl.ds(j * 16, 16)] for j in range(nj)]
        for i in range(D_MODEL):
            xi = splat(cvec_v, i)
            for j in range(nj):
                oacc[j] = oacc[j] + xi * w_v[i, pl.ds(j * 16, 16)]
        for i in range(2 * D_MODEL):
            xi = splat(mq_v, i)
            for j in range(nj):
                oacc[j] = oacc[j] + xi * w_v[D_MODEL + i, pl.ds(j * 16, 16)]
        for j in range(nj):
            x = oacc[j]
            en = jnp.exp(-2.0 * jnp.abs(x))
            t = (1.0 - en) / (1.0 + en)
            out_v[pl.ds(j * 16, 16)] = jnp.sign(x) * t
        pltpu.sync_copy(out_v, out_hbm)


def _fusion_sc(crystal_keys, top_vals, top_idx, qvec, morph, W_fusion,
               b_fusion):
    mesh = plsc.VectorSubcoreMesh(core_axis_name="c", subcore_axis_name="s")
    fin = 2 * D_MODEL + D_MODEL
    run = pl.kernel(
        _stage_c_body,
        mesh=mesh,
        compiler_params=pltpu.CompilerParams(
            needs_layout_passes=False, use_tc_tiling_on_sc=False),
        out_type=jax.ShapeDtypeStruct((D_MODEL,), jnp.float32),
        scratch_types=[
            pltpu.VMEM((K,), jnp.int32),
            pltpu.VMEM((16 + K,), jnp.float32),
            pltpu.VMEM((K, D_MODEL), jnp.float32),
            pltpu.VMEM((16 + 2 * D_MODEL,), jnp.float32),
            pltpu.VMEM((fin, D_MODEL), jnp.float32),
            pltpu.VMEM((D_MODEL,), jnp.float32),
            pltpu.VMEM((D_MODEL,), jnp.float32),
            pltpu.VMEM((16 + D_MODEL,), jnp.float32),
            pltpu.SemaphoreType.DMA,
        ],
    )
    return run(crystal_keys, top_vals, top_idx, qvec, morph, W_fusion,
               b_fusion)


def kernel(query_embedding, crystal_keys, morph_context, W_fusion, b_fusion,
           top_k):
    del top_k
    q = query_embedding.mean(axis=0)
    qn = q / (jnp.linalg.norm(q) + 1e-8)
    qnb = qn.astype(jnp.bfloat16).reshape(1, D_MODEL)

    bmax = pl.pallas_call(
        _stage_a_body,
        grid=(N_CHUNKS,),
        in_specs=[
            pl.BlockSpec((1, D_MODEL), lambda i: (0, 0)),
            pl.BlockSpec((CHUNK, D_MODEL), lambda i: (i, 0)),
        ],
        out_specs=pl.BlockSpec((1, 1, BLK_PER_CHUNK), lambda i: (i, 0, 0)),
        out_shape=jax.ShapeDtypeStruct((N_CHUNKS, 1, BLK_PER_CHUNK),
                                       jnp.float32),
    )(qnb, crystal_keys)

    sel = pl.pallas_call(
        _stage_s_body,
        out_shape=jax.ShapeDtypeStruct((1, NSEL), jnp.int32),
    )(bmax.reshape(1, N_BLKS))

    keys_blk = crystal_keys.reshape(N_BLKS, BLK, D_MODEL)
    top_vals, top_idx = pl.pallas_call(
        _stage_b_body,
        grid_spec=pltpu.PrefetchScalarGridSpec(
            num_scalar_prefetch=1,
            grid=(NSEL,),
            in_specs=[
                pl.BlockSpec((1, D_MODEL), lambda i, sel: (0, 0)),
                pl.BlockSpec((1, BLK, D_MODEL), lambda i, sel: (sel[i], 0, 0)),
            ],
            out_specs=[
                pl.BlockSpec((1, K), lambda i, sel: (0, 0)),
                pl.BlockSpec((1, K), lambda i, sel: (0, 0)),
            ],
            scratch_shapes=[pltpu.VMEM((NSEL, BLK), jnp.float32)],
        ),
        out_shape=[
            jax.ShapeDtypeStruct((1, K), jnp.float32),
            jax.ShapeDtypeStruct((1, K), jnp.int32),
        ],
    )(sel.reshape(NSEL), qnb, keys_blk)

    tv = top_vals.reshape(K)
    ti = top_idx.reshape(K)
    fused = _fusion_sc(crystal_keys, tv, ti, q, morph_context, W_fusion,
                       b_fusion)
    return fused, tv, ti


# speedup vs baseline: 1.8675x; 1.8675x over previous
import functools

import jax
import jax.numpy as jnp
from jax import lax
from jax.experimental import pallas as pl
from jax.experimental.pallas import tpu as pltpu
from jax.experimental.pallas import tpu_sc as plsc

D_MODEL = 64
N_KEYS = 1000000
CHUNK = 20000
N_CHUNKS = N_KEYS // CHUNK
BLK = 1000
N_BLKS = N_KEYS // BLK
BLK_PER_CHUNK = CHUNK // BLK
K = 16
NSEL = 24

_NEG = float("-inf")
_IMAX = 2**31 - 1


def _sims_block(qnb, x):
    sumsq = jnp.sum(x * x, axis=1, keepdims=True)
    kn = x / (jnp.sqrt(sumsq) + 1e-8)
    knb = kn.astype(jnp.bfloat16)
    return lax.dot_general(
        qnb, knb, dimension_numbers=(((1,), (1,)), ((), ())),
        preferred_element_type=jnp.float32)


def _stage_a_body(qnb_ref, keys_ref, bmax_ref):
    dots = _sims_block(qnb_ref[...], keys_ref[...])
    iota_b = lax.broadcasted_iota(jnp.int32, (1, BLK_PER_CHUNK), 1)
    bmax = jnp.zeros((1, BLK_PER_CHUNK), jnp.float32)
    for b in range(BLK_PER_CHUNK):
        mb = jnp.max(dots[:, b * BLK:(b + 1) * BLK])
        bmax = jnp.where(iota_b == b, mb, bmax)
    bmax_ref[...] = bmax.reshape(1, 1, BLK_PER_CHUNK)


def _stage_s_body(cmax_ref, sel_ref):
    work = cmax_ref[...]
    iota = lax.broadcasted_iota(jnp.int32, (1, N_BLKS), 1)
    iota_s = lax.broadcasted_iota(jnp.int32, (1, NSEL), 1)
    sel = jnp.zeros((1, NSEL), jnp.int32)
    for t in range(NSEL):
        m = jnp.max(work)
        pos = jnp.min(jnp.where(work == m, iota, _IMAX))
        sel = jnp.where(iota_s == t, pos, sel)
        work = jnp.where(iota == pos, _NEG, work)
    sel_ref[...] = sel


def _stage_b_body(sel_ref, qnb_ref, keys_ref, tv_ref, ti_ref, cand_scr):
    i = pl.program_id(0)
    x = keys_ref[...].reshape(BLK, D_MODEL)
    cand_scr[pl.ds(i, 1), :] = _sims_block(qnb_ref[...], x)

    @pl.when(i == NSEL - 1)
    def _():
        work = cand_scr[...]
        iota_sc = lax.broadcasted_iota(jnp.int32, (NSEL, 1), 0)
        sel_col = jnp.zeros((NSEL, 1), jnp.int32)
        for t in range(NSEL):
            sel_col = jnp.where(iota_sc == t, sel_ref[t], sel_col)
        gidx = (sel_col * BLK
                + lax.broadcasted_iota(jnp.int32, (NSEL, BLK), 1))
        iota_k = lax.broadcasted_iota(jnp.int32, (1, K), 1)
        vals_vec = jnp.zeros((1, K), jnp.float32)
        idx_vec = jnp.zeros((1, K), jnp.int32)
        for t in range(K):
            m = jnp.max(work)
            tie = work == m
            bidx = jnp.min(jnp.where(tie, gidx, _IMAX))
            vals_vec = jnp.where(iota_k == t, m, vals_vec)
            idx_vec = jnp.where(iota_k == t, bidx, idx_vec)
            work = jnp.where(tie & (gidx == bidx), _NEG, work)
        tv_ref[...] = vals_vec
        ti_ref[...] = idx_vec


def _stage_c_body(keys_hbm, vals_hbm, idx_hbm, q_hbm, morph_hbm, w_hbm,
                  b_hbm, out_hbm, idx_v, vals_v, rows_v, mq_v, w_v, b_v,
                  out_v, cvec_v, sem):
    core = lax.axis_index("c")
    sub = lax.axis_index("s")

    @pl.when((core == 0) & (sub == 0))
    def _():
        pltpu.sync_copy(idx_hbm, idx_v)
        pltpu.sync_copy(vals_hbm, vals_v.at[pl.ds(16, K)])
        pltpu.async_copy(keys_hbm.at[idx_v], rows_v, sem).wait()
        pltpu.sync_copy(w_hbm, w_v)
        pltpu.sync_copy(b_hbm, b_v)
        pltpu.sync_copy(morph_hbm, mq_v.at[pl.ds(16, D_MODEL)])
        pltpu.sync_copy(q_hbm, mq_v.at[pl.ds(16 + D_MODEL, D_MODEL)])

        def splat(ref, i):
            return plsc.load_gather(ref, [jnp.full((16,), 16 + i,
                                                   jnp.int32)])

        m = splat(vals_v, 0)
        for k in range(1, K):
            m = jnp.maximum(m, splat(vals_v, k))
        s = jnp.zeros((16,), jnp.float32)
        for k in range(K):
            s = s + jnp.exp(splat(vals_v, k) - m)

        nj = D_MODEL // 16
        cacc = [jnp.zeros((16,), jnp.float32) for _ in range(nj)]
        for k in range(K):
            wk = jnp.exp(splat(vals_v, k) - m) / s
            for j in range(nj):
                cacc[j] = cacc[j] + wk * rows_v[k, pl.ds(j * 16, 16)]
        for j in range(nj):
            out_v[pl.ds(j * 16, 16)] = cacc[j]
        pltpu.sync_copy(out_v, out_hbm)
        pltpu.sync_copy(out_hbm, cvec_v.at[pl.ds(16, D_MODEL)])

        oacc = [b_v[pl.ds(j * 16, 16)] for j in range(nj)]
        for i in range(D_MODEL):
            xi = splat(cvec_v, i)
            for j in range(nj):
                oacc[j] = oacc[j] + xi * w_v[i, pl.ds(j * 16, 16)]
        for i in range(2 * D_MODEL):
            xi = splat(mq_v, i)
            for j in range(nj):
                oacc[j] = oacc[j] + xi * w_v[D_MODEL + i, pl.ds(j * 16, 16)]
        for j in range(nj):
            x = oacc[j]
            en = jnp.exp(-2.0 * jnp.abs(x))
            t = (1.0 - en) / (1.0 + en)
            out_v[pl.ds(j * 16, 16)] = jnp.sign(x) * t
        pltpu.sync_copy(out_v, out_hbm)


def _fusion_sc(crystal_keys, top_vals, top_idx, qvec, morph, W_fusion,
               b_fusion):
    mesh = plsc.VectorSubcoreMesh(core_axis_name="c", subcore_axis_name="s")
    fin = 2 * D_MODEL + D_MODEL
    run = pl.kernel(
        _stage_c_body,
        mesh=mesh,
        compiler_params=pltpu.CompilerParams(
            needs_layout_passes=False, use_tc_tiling_on_sc=False),
        out_type=jax.ShapeDtypeStruct((D_MODEL,), jnp.float32),
        scratch_types=[
            pltpu.VMEM((K,), jnp.int32),
            pltpu.VMEM((16 + K,), jnp.float32),
            pltpu.VMEM((K, D_MODEL), jnp.float32),
            pltpu.VMEM((16 + 2 * D_MODEL,), jnp.float32),
            pltpu.VMEM((fin, D_MODEL), jnp.float32),
            pltpu.VMEM((D_MODEL,), jnp.float32),
            pltpu.VMEM((D_MODEL,), jnp.float32),
            pltpu.VMEM((16 + D_MODEL,), jnp.float32),
            pltpu.SemaphoreType.DMA,
        ],
    )
    return run(crystal_keys, top_vals, top_idx, qvec, morph, W_fusion,
               b_fusion)


def kernel(query_embedding, crystal_keys, morph_context, W_fusion, b_fusion,
           top_k):
    del top_k
    q = query_embedding.mean(axis=0)
    qn = q / (jnp.linalg.norm(q) + 1e-8)
    qnb = qn.astype(jnp.bfloat16).reshape(1, D_MODEL)

    bmax = pl.pallas_call(
        _stage_a_body,
        grid=(N_CHUNKS,),
        in_specs=[
            pl.BlockSpec((1, D_MODEL), lambda i: (0, 0)),
            pl.BlockSpec((CHUNK, D_MODEL), lambda i: (i, 0)),
        ],
        out_specs=pl.BlockSpec((1, 1, BLK_PER_CHUNK), lambda i: (i, 0, 0)),
        out_shape=jax.ShapeDtypeStruct((N_CHUNKS, 1, BLK_PER_CHUNK),
                                       jnp.float32),
    )(qnb, crystal_keys)

    sel = pl.pallas_call(
        _stage_s_body,
        out_shape=jax.ShapeDtypeStruct((1, NSEL), jnp.int32),
    )(bmax.reshape(1, N_BLKS))

    keys_blk = crystal_keys.reshape(N_BLKS, BLK, D_MODEL)
    top_vals, top_idx = pl.pallas_call(
        _stage_b_body,
        grid_spec=pltpu.PrefetchScalarGridSpec(
            num_scalar_prefetch=1,
            grid=(NSEL,),
            in_specs=[
                pl.BlockSpec((1, D_MODEL), lambda i, sel: (0, 0)),
                pl.BlockSpec((1, BLK, D_MODEL), lambda i, sel: (sel[i], 0, 0)),
            ],
            out_specs=[
                pl.BlockSpec((1, K), lambda i, sel: (0, 0)),
                pl.BlockSpec((1, K), lambda i, sel: (0, 0)),
            ],
            scratch_shapes=[pltpu.VMEM((NSEL, BLK), jnp.float32)],
        ),
        out_shape=[
            jax.ShapeDtypeStruct((1, K), jnp.float32),
            jax.ShapeDtypeStruct((1, K), jnp.int32),
        ],
    )(sel.reshape(NSEL), qnb, keys_blk)

    tv = top_vals.reshape(K)
    ti = top_idx.reshape(K)
    fused = _fusion_sc(crystal_keys, tv, ti, q, morph_context, W_fusion,
                       b_fusion)
    return fused, tv, ti
